# baseline (speedup 1.0000x reference)
import jax
import jax.numpy as jnp
from jax import lax
from jax.experimental import pallas as pl
from jax.experimental.pallas import tpu as pltpu
from jax.experimental.pallas import tpu_sc as plsc

N_NODES = 50000
N_EDGES = 800000
F = 64
HALF = 32
LOG2 = 0.6931471805599453

NSUB = 16
NPAD = 50176
NODE_STRIPE = NPAD // NSUB
NODE_HCHUNK = NODE_STRIPE // 2
NODE_WCHUNK = 448
E_PAD = 819200
E_PER_TILE = E_PAD // NSUB
E_CHUNK = 2048
QROWS = 256
NQ = E_CHUNK // QROWS
N_STEPS = E_PER_TILE // E_CHUNK

EBLK = 3200
NBLK = 2000


def _ssp(x):
    return jax.nn.softplus(x) - LOG2


def _edge_mlp_body(bf_ref, eh_ref, w1_ref, b1_ref, w2_ref, b2_ref, out_ref):
    h = _ssp(jnp.dot(bf_ref[...], w1_ref[...],
                     preferred_element_type=jnp.float32) + b1_ref[...])
    h = _ssp(jnp.dot(h, w2_ref[...],
                     preferred_element_type=jnp.float32) + b2_ref[...])
    m = h * eh_ref[...]
    q4 = EBLK // 4
    for half in range(2):
        mh = m[:, half * HALF:(half + 1) * HALF]
        out_ref[half] = jnp.concatenate(
            [mh[g * q4:(g + 1) * q4] for g in range(4)], axis=1)


def _sc_scatter_body(dst_ref, m_ref, sums_ref, cnt_ref,
                     idx_v, rows_v, ones_v, cstage_v, sum_sh, cnt_sh,
                     sem0, sem1):
    cid = lax.axis_index("c")
    sid = lax.axis_index("s")
    r0 = sid * NODE_STRIPE
    z16 = jnp.zeros((16,), jnp.float32)
    sems = (sem0, sem1)

    for j in range(8):
        ones_v[pl.ds(j * 16, 16)] = jnp.ones((16,), jnp.float32)

    def zrow(i, carry):
        rows_v[0, i, pl.ds(0, 16)] = z16
        rows_v[0, i, pl.ds(16, 16)] = z16
        return carry
    lax.fori_loop(0, QROWS, zrow, 0)

    def zcnt(i, carry):
        cstage_v[pl.ds(i * 16, 16)] = z16
        return carry
    lax.fori_loop(0, NODE_HCHUNK // 16, zcnt, 0)

    for t in range(NODE_STRIPE // QROWS):
        pltpu.sync_copy(rows_v.at[0],
                        sum_sh.at[pl.ds(r0 + t * QROWS, QROWS)])
    pltpu.sync_copy(
        rows_v.at[0, pl.ds(0, NODE_STRIPE % QROWS)],
        sum_sh.at[pl.ds(r0 + QROWS * (NODE_STRIPE // QROWS),
                        NODE_STRIPE % QROWS)])
    for t in range(2):
        pltpu.sync_copy(cstage_v,
                        cnt_sh.at[pl.ds(r0 + t * NODE_HCHUNK, NODE_HCHUNK)])
    plsc.subcore_barrier()

    ebase = sid * E_PER_TILE
    ibase = sid * (E_PER_TILE // 128)

    def mload(k, q, buf):
        pltpu.async_copy(
            m_ref.at[cid, pl.ds(ebase + k * E_CHUNK + q * QROWS, QROWS)],
            rows_v.at[buf], sems[buf])

    mload(0, 0, 0)

    def step(k, carry):
        pltpu.sync_copy(dst_ref.at[pl.ds(ibase + k * (E_CHUNK // 128),
                                         E_CHUNK // 128)], idx_v)
        for q in range(NQ):
            b = q % 2
            nb = (q + 1) % 2
            if q < NQ - 1:
                mload(k, q + 1, nb)
            else:
                @pl.when(k < N_STEPS - 1)
                def _():
                    mload(k + 1, 0, nb)
            pltpu.make_async_copy(m_ref.at[cid, pl.ds(0, QROWS)],
                                  rows_v.at[b], sems[b]).wait()
            for j in range(QROWS // 128):
                pltpu.sync_copy(rows_v.at[b, pl.ds(j * 128, 128)],
                                sum_sh.at[idx_v.at[q * 2 + j]], add=True)

            @pl.when(cid == q // (NQ // 2))
            def _():
                for j in range(QROWS // 128):
                    pltpu.sync_copy(ones_v,
                                    cnt_sh.at[idx_v.at[q * 2 + j]], add=True)
        return carry

    lax.fori_loop(0, N_STEPS, step, 0)
    plsc.subcore_barrier()

    for t in range(NODE_STRIPE // QROWS):
        s0 = r0 + t * QROWS
        pltpu.sync_copy(sum_sh.at[pl.ds(s0, QROWS)], rows_v.at[0])
        pltpu.sync_copy(rows_v.at[0], sums_ref.at[cid, pl.ds(s0, QROWS)])
    s0 = r0 + QROWS * (NODE_STRIPE // QROWS)
    tail = NODE_STRIPE % QROWS
    pltpu.sync_copy(sum_sh.at[pl.ds(s0, tail)], rows_v.at[0, pl.ds(0, tail)])
    pltpu.sync_copy(rows_v.at[0, pl.ds(0, tail)],
                    sums_ref.at[cid, pl.ds(s0, tail)])
    for t in range(2):
        s0 = r0 + t * NODE_HCHUNK
        pltpu.sync_copy(cnt_sh.at[pl.ds(s0, NODE_HCHUNK)], cstage_v)
        pltpu.sync_copy(cstage_v, cnt_ref.at[cid, pl.ds(s0, NODE_HCHUNK)])


def _node_mlp_body(s_ref, c_ref, w3_ref, b3_ref, out_ref):
    s = s_ref[...]
    cnt = c_ref[0] + c_ref[1]
    inv = 1.0 / jnp.maximum(cnt, 1.0)
    node = jnp.concatenate([s[0], s[1]], axis=1) * inv
    out_ref[...] = _ssp(jnp.dot(node, w3_ref[...],
                                preferred_element_type=jnp.float32)
                        + b3_ref[...])


def kernel(bf, eh, edge_index, W1, b1, W2, b2, W3, b3):
    f32 = jnp.float32
    bf = bf.astype(f32)
    eh = eh.astype(f32)

    m_all = pl.pallas_call(
        _edge_mlp_body,
        grid=(N_EDGES // EBLK,),
        in_specs=[
            pl.BlockSpec((EBLK, F), lambda i: (i, 0)),
            pl.BlockSpec((EBLK, F), lambda i: (i, 0)),
            pl.BlockSpec((F, F), lambda i: (0, 0)),
            pl.BlockSpec((1, F), lambda i: (0, 0)),
            pl.BlockSpec((F, F), lambda i: (0, 0)),
            pl.BlockSpec((1, F), lambda i: (0, 0)),
        ],
        out_specs=pl.BlockSpec((2, EBLK // 4, 128), lambda i: (0, i, 0)),
        out_shape=jax.ShapeDtypeStruct((2, E_PAD // 4, 128), f32),
    )(bf, eh, W1.astype(f32), b1.astype(f32).reshape(1, F),
      W2.astype(f32), b2.astype(f32).reshape(1, F))
    m_all = m_all.reshape(2, E_PAD, HALF)

    dst = edge_index[1].astype(jnp.int32)
    n_extra = E_PAD - N_EDGES
    pad_dst = N_NODES + (jnp.arange(n_extra, dtype=jnp.int32)
                         % (NPAD - N_NODES))
    dst_perm = (jnp.concatenate([dst, pad_dst])
                .reshape(E_PAD // EBLK, 4, EBLK // 4)
                .transpose(0, 2, 1).reshape(-1))
    dst2 = dst_perm.reshape(E_PAD // 128, 128)

    mesh = plsc.VectorSubcoreMesh(core_axis_name="c", subcore_axis_name="s")
    sums, cnts = pl.kernel(
        _sc_scatter_body,
        out_type=[jax.ShapeDtypeStruct((2, NPAD, HALF), f32),
                  jax.ShapeDtypeStruct((2, NPAD), f32)],
        mesh=mesh,
        scratch_types=[
            pltpu.VMEM((E_CHUNK // 128, 128), jnp.int32),
            pltpu.VMEM((2, QROWS, HALF), f32),
            pltpu.VMEM((128,), f32),
            pltpu.VMEM((NODE_HCHUNK,), f32),
            pltpu.VMEM_SHARED((NPAD, HALF), f32),
            pltpu.VMEM_SHARED((NPAD,), f32),
            pltpu.SemaphoreType.DMA,
            pltpu.SemaphoreType.DMA,
        ],
        compiler_params=pltpu.CompilerParams(use_tc_tiling_on_sc=False),
    )(dst2, m_all)

    out = pl.pallas_call(
        _node_mlp_body,
        grid=(N_NODES // NBLK,),
        in_specs=[
            pl.BlockSpec((2, NBLK, HALF), lambda i: (0, i, 0)),
            pl.BlockSpec((2, NBLK, 1), lambda i: (0, i, 0)),
            pl.BlockSpec((F, F), lambda i: (0, 0)),
            pl.BlockSpec((1, F), lambda i: (0, 0)),
        ],
        out_specs=pl.BlockSpec((NBLK, F), lambda i: (i, 0)),
        out_shape=jax.ShapeDtypeStruct((N_NODES, F), f32),
    )(sums, cnts.reshape(2, NPAD, 1), W3.astype(f32),
      b3.astype(f32).reshape(1, F))
    return out


# speedup vs baseline: 1.0115x; 1.0115x over previous
import jax
import jax.numpy as jnp
from jax import lax
from jax.experimental import pallas as pl
from jax.experimental.pallas import tpu as pltpu
from jax.experimental.pallas import tpu_sc as plsc

N_NODES = 50000
N_EDGES = 800000
F = 64
HALF = 32
LOG2 = 0.6931471805599453

NSUB = 16
NPAD = 50176
NODE_STRIPE = NPAD // NSUB
NODE_HCHUNK = NODE_STRIPE // 2
NODE_WCHUNK = 448
E_PAD = 819200
E_PER_TILE = E_PAD // NSUB
E_CHUNK = 2048
QROWS = 256
NQ = E_CHUNK // QROWS
N_STEPS = E_PER_TILE // E_CHUNK

EBLK = 6400
NBLK = 2000


def _ssp(x):
    return jax.nn.softplus(x) - LOG2


def _edge_mlp_body(bf_ref, eh_ref, w1_ref, b1_ref, w2_ref, b2_ref, out_ref):
    h = _ssp(jnp.dot(bf_ref[...], w1_ref[...],
                     preferred_element_type=jnp.float32) + b1_ref[...])
    h = _ssp(jnp.dot(h, w2_ref[...],
                     preferred_element_type=jnp.float32) + b2_ref[...])
    m = h * eh_ref[...]
    q4 = EBLK // 4
    for half in range(2):
        mh = m[:, half * HALF:(half + 1) * HALF]
        out_ref[half] = jnp.concatenate(
            [mh[g * q4:(g + 1) * q4] for g in range(4)], axis=1)


def _sc_scatter_body(dst_ref, m_ref, sums_ref, cnt_ref,
                     idx_v, rows_v, ones_v, cstage_v, sum_sh, cnt_sh,
                     sem0, sem1):
    cid = lax.axis_index("c")
    sid = lax.axis_index("s")
    r0 = sid * NODE_STRIPE
    z16 = jnp.zeros((16,), jnp.float32)
    sems = (sem0, sem1)

    for j in range(8):
        ones_v[pl.ds(j * 16, 16)] = jnp.ones((16,), jnp.float32)

    def zrow(i, carry):
        rows_v[0, i, pl.ds(0, 16)] = z16
        rows_v[0, i, pl.ds(16, 16)] = z16
        return carry
    lax.fori_loop(0, QROWS, zrow, 0)

    def zcnt(i, carry):
        cstage_v[pl.ds(i * 16, 16)] = z16
        return carry
    lax.fori_loop(0, NODE_HCHUNK // 16, zcnt, 0)

    for t in range(NODE_STRIPE // QROWS):
        pltpu.sync_copy(rows_v.at[0],
                        sum_sh.at[pl.ds(r0 + t * QROWS, QROWS)])
    pltpu.sync_copy(
        rows_v.at[0, pl.ds(0, NODE_STRIPE % QROWS)],
        sum_sh.at[pl.ds(r0 + QROWS * (NODE_STRIPE // QROWS),
                        NODE_STRIPE % QROWS)])
    for t in range(2):
        pltpu.sync_copy(cstage_v,
                        cnt_sh.at[pl.ds(r0 + t * NODE_HCHUNK, NODE_HCHUNK)])
    plsc.subcore_barrier()

    ebase = sid * E_PER_TILE
    ibase = sid * (E_PER_TILE // 128)

    def mload(k, q, buf):
        pltpu.async_copy(
            m_ref.at[cid, pl.ds(ebase + k * E_CHUNK + q * QROWS, QROWS)],
            rows_v.at[buf], sems[buf])

    mload(0, 0, 0)

    def step(k, carry):
        pltpu.sync_copy(dst_ref.at[pl.ds(ibase + k * (E_CHUNK // 128),
                                         E_CHUNK // 128)], idx_v)
        for q in range(NQ):
            b = q % 2
            nb = (q + 1) % 2
            if q < NQ - 1:
                mload(k, q + 1, nb)
            else:
                @pl.when(k < N_STEPS - 1)
                def _():
                    mload(k + 1, 0, nb)
            pltpu.make_async_copy(m_ref.at[cid, pl.ds(0, QROWS)],
                                  rows_v.at[b], sems[b]).wait()
            for j in range(QROWS // 128):
                pltpu.sync_copy(rows_v.at[b, pl.ds(j * 128, 128)],
                                sum_sh.at[idx_v.at[q * 2 + j]], add=True)

            @pl.when(cid == q // (NQ // 2))
            def _():
                for j in range(QROWS // 128):
                    pltpu.sync_copy(ones_v,
                                    cnt_sh.at[idx_v.at[q * 2 + j]], add=True)
        return carry

    lax.fori_loop(0, N_STEPS, step, 0)
    plsc.subcore_barrier()

    for t in range(NODE_STRIPE // QROWS):
        s0 = r0 + t * QROWS
        pltpu.sync_copy(sum_sh.at[pl.ds(s0, QROWS)], rows_v.at[0])
        pltpu.sync_copy(rows_v.at[0], sums_ref.at[cid, pl.ds(s0, QROWS)])
    s0 = r0 + QROWS * (NODE_STRIPE // QROWS)
    tail = NODE_STRIPE % QROWS
    pltpu.sync_copy(sum_sh.at[pl.ds(s0, tail)], rows_v.at[0, pl.ds(0, tail)])
    pltpu.sync_copy(rows_v.at[0, pl.ds(0, tail)],
                    sums_ref.at[cid, pl.ds(s0, tail)])
    for t in range(2):
        s0 = r0 + t * NODE_HCHUNK
        pltpu.sync_copy(cnt_sh.at[pl.ds(s0, NODE_HCHUNK)], cstage_v)
        pltpu.sync_copy(cstage_v, cnt_ref.at[cid, pl.ds(s0, NODE_HCHUNK)])


def _node_mlp_body(s_ref, c_ref, w3_ref, b3_ref, out_ref):
    s = s_ref[...]
    cnt = c_ref[0] + c_ref[1]
    inv = 1.0 / jnp.maximum(cnt, 1.0)
    node = jnp.concatenate([s[0], s[1]], axis=1) * inv
    out_ref[...] = _ssp(jnp.dot(node, w3_ref[...],
                                preferred_element_type=jnp.float32)
                        + b3_ref[...])


def kernel(bf, eh, edge_index, W1, b1, W2, b2, W3, b3):
    f32 = jnp.float32
    bf = bf.astype(f32)
    eh = eh.astype(f32)

    m_all = pl.pallas_call(
        _edge_mlp_body,
        grid=(N_EDGES // EBLK,),
        in_specs=[
            pl.BlockSpec((EBLK, F), lambda i: (i, 0)),
            pl.BlockSpec((EBLK, F), lambda i: (i, 0)),
            pl.BlockSpec((F, F), lambda i: (0, 0)),
            pl.BlockSpec((1, F), lambda i: (0, 0)),
            pl.BlockSpec((F, F), lambda i: (0, 0)),
            pl.BlockSpec((1, F), lambda i: (0, 0)),
        ],
        out_specs=pl.BlockSpec((2, EBLK // 4, 128), lambda i: (0, i, 0)),
        out_shape=jax.ShapeDtypeStruct((2, E_PAD // 4, 128), f32),
    )(bf, eh, W1.astype(f32), b1.astype(f32).reshape(1, F),
      W2.astype(f32), b2.astype(f32).reshape(1, F))
    m_all = m_all.reshape(2, E_PAD, HALF)

    dst = edge_index[1].astype(jnp.int32)
    n_extra = E_PAD - N_EDGES
    pad_dst = N_NODES + (jnp.arange(n_extra, dtype=jnp.int32)
                         % (NPAD - N_NODES))
    dst_perm = (jnp.concatenate([dst, pad_dst])
                .reshape(E_PAD // EBLK, 4, EBLK // 4)
                .transpose(0, 2, 1).reshape(-1))
    dst2 = dst_perm.reshape(E_PAD // 128, 128)

    mesh = plsc.VectorSubcoreMesh(core_axis_name="c", subcore_axis_name="s")
    sums, cnts = pl.kernel(
        _sc_scatter_body,
        out_type=[jax.ShapeDtypeStruct((2, NPAD, HALF), f32),
                  jax.ShapeDtypeStruct((2, NPAD), f32)],
        mesh=mesh,
        scratch_types=[
            pltpu.VMEM((E_CHUNK // 128, 128), jnp.int32),
            pltpu.VMEM((2, QROWS, HALF), f32),
            pltpu.VMEM((128,), f32),
            pltpu.VMEM((NODE_HCHUNK,), f32),
            pltpu.VMEM_SHARED((NPAD, HALF), f32),
            pltpu.VMEM_SHARED((NPAD,), f32),
            pltpu.SemaphoreType.DMA,
            pltpu.SemaphoreType.DMA,
        ],
        compiler_params=pltpu.CompilerParams(use_tc_tiling_on_sc=False),
    )(dst2, m_all)

    out = pl.pallas_call(
        _node_mlp_body,
        grid=(N_NODES // NBLK,),
        in_specs=[
            pl.BlockSpec((2, NBLK, HALF), lambda i: (0, i, 0)),
            pl.BlockSpec((2, NBLK, 1), lambda i: (0, i, 0)),
            pl.BlockSpec((F, F), lambda i: (0, 0)),
            pl.BlockSpec((1, F), lambda i: (0, 0)),
        ],
        out_specs=pl.BlockSpec((NBLK, F), lambda i: (i, 0)),
        out_shape=jax.ShapeDtypeStruct((N_NODES, F), f32),
    )(sums, cnts.reshape(2, NPAD, 1), W3.astype(f32),
      b3.astype(f32).reshape(1, F))
    return out


# speedup vs baseline: 1.5923x; 1.5742x over previous
import jax
import jax.numpy as jnp
from jax import lax
from jax.experimental import pallas as pl
from jax.experimental.pallas import tpu as pltpu
from jax.experimental.pallas import tpu_sc as plsc

N_NODES = 50000
N_EDGES = 800000
F = 64
HALF = 32
LOG2 = 0.6931471805599453

NSUB = 16
NPAD = 50176
NODE_STRIPE = NPAD // NSUB
NODE_HCHUNK = NODE_STRIPE // 2
NODE_WCHUNK = 448
E_PAD = 819200
E_PER_TILE = E_PAD // NSUB
E_CHUNK = 2048
QROWS = 256
NQ = E_CHUNK // QROWS
N_STEPS = E_PER_TILE // E_CHUNK

EBLK = 6400
NBLK = 2000


def _ssp(x):
    return jax.nn.softplus(x) - LOG2


def _edge_mlp_body(bft_ref, eht_ref, w1t_ref, b1_ref, w2t_ref, b2_ref,
                   out_ref):
    ht = _ssp(jnp.dot(w1t_ref[...], bft_ref[...],
                      preferred_element_type=jnp.float32) + b1_ref[...])
    ht = _ssp(jnp.dot(w2t_ref[...], ht,
                      preferred_element_type=jnp.float32) + b2_ref[...])
    mt = ht * eht_ref[...]
    m = mt.T
    q4 = EBLK // 4
    for half in range(2):
        mh = m[:, half * HALF:(half + 1) * HALF]
        out_ref[half] = jnp.concatenate(
            [mh[g * q4:(g + 1) * q4] for g in range(4)], axis=1)


def _sc_scatter_body(dst_ref, m_ref, sums_ref, cnt_ref,
                     idx_v, rows_v, ones_v, cstage_v, sum_sh, cnt_sh,
                     sem0, sem1):
    cid = lax.axis_index("c")
    sid = lax.axis_index("s")
    r0 = sid * NODE_STRIPE
    z16 = jnp.zeros((16,), jnp.float32)
    sems = (sem0, sem1)

    for j in range(8):
        ones_v[pl.ds(j * 16, 16)] = jnp.ones((16,), jnp.float32)

    def zrow(i, carry):
        rows_v[0, i, pl.ds(0, 16)] = z16
        rows_v[0, i, pl.ds(16, 16)] = z16
        return carry
    lax.fori_loop(0, QROWS, zrow, 0)

    def zcnt(i, carry):
        cstage_v[pl.ds(i * 16, 16)] = z16
        return carry
    lax.fori_loop(0, NODE_HCHUNK // 16, zcnt, 0)

    for t in range(NODE_STRIPE // QROWS):
        pltpu.sync_copy(rows_v.at[0],
                        sum_sh.at[pl.ds(r0 + t * QROWS, QROWS)])
    pltpu.sync_copy(
        rows_v.at[0, pl.ds(0, NODE_STRIPE % QROWS)],
        sum_sh.at[pl.ds(r0 + QROWS * (NODE_STRIPE // QROWS),
                        NODE_STRIPE % QROWS)])
    for t in range(2):
        pltpu.sync_copy(cstage_v,
                        cnt_sh.at[pl.ds(r0 + t * NODE_HCHUNK, NODE_HCHUNK)])
    plsc.subcore_barrier()

    ebase = sid * E_PER_TILE
    ibase = sid * (E_PER_TILE // 128)

    def mload(k, q, buf):
        pltpu.async_copy(
            m_ref.at[cid, pl.ds(ebase + k * E_CHUNK + q * QROWS, QROWS)],
            rows_v.at[buf], sems[buf])

    mload(0, 0, 0)

    def step(k, carry):
        pltpu.sync_copy(dst_ref.at[pl.ds(ibase + k * (E_CHUNK // 128),
                                         E_CHUNK // 128)], idx_v)
        for q in range(NQ):
            b = q % 2
            nb = (q + 1) % 2
            if q < NQ - 1:
                mload(k, q + 1, nb)
            else:
                @pl.when(k < N_STEPS - 1)
                def _():
                    mload(k + 1, 0, nb)
            pltpu.make_async_copy(m_ref.at[cid, pl.ds(0, QROWS)],
                                  rows_v.at[b], sems[b]).wait()
            for j in range(QROWS // 128):
                pltpu.sync_copy(rows_v.at[b, pl.ds(j * 128, 128)],
                                sum_sh.at[idx_v.at[q * 2 + j]], add=True)

            @pl.when(cid == q // (NQ // 2))
            def _():
                for j in range(QROWS // 128):
                    pltpu.sync_copy(ones_v,
                                    cnt_sh.at[idx_v.at[q * 2 + j]], add=True)
        return carry

    lax.fori_loop(0, N_STEPS, step, 0)
    plsc.subcore_barrier()

    for t in range(NODE_STRIPE // QROWS):
        s0 = r0 + t * QROWS
        pltpu.sync_copy(sum_sh.at[pl.ds(s0, QROWS)], rows_v.at[0])
        pltpu.sync_copy(rows_v.at[0], sums_ref.at[cid, pl.ds(s0, QROWS)])
    s0 = r0 + QROWS * (NODE_STRIPE // QROWS)
    tail = NODE_STRIPE % QROWS
    pltpu.sync_copy(sum_sh.at[pl.ds(s0, tail)], rows_v.at[0, pl.ds(0, tail)])
    pltpu.sync_copy(rows_v.at[0, pl.ds(0, tail)],
                    sums_ref.at[cid, pl.ds(s0, tail)])
    for t in range(2):
        s0 = r0 + t * NODE_HCHUNK
        pltpu.sync_copy(cnt_sh.at[pl.ds(s0, NODE_HCHUNK)], cstage_v)
        pltpu.sync_copy(cstage_v, cnt_ref.at[cid, pl.ds(s0, NODE_HCHUNK)])


def _node_mlp_body(s_ref, c_ref, w3_ref, b3_ref, out_ref):
    s = s_ref[...]
    cnt = c_ref[0] + c_ref[1]
    inv = 1.0 / jnp.maximum(cnt, 1.0)
    node = jnp.concatenate([s[0], s[1]], axis=1) * inv
    out_ref[...] = _ssp(jnp.dot(node, w3_ref[...],
                                preferred_element_type=jnp.float32)
                        + b3_ref[...])


def kernel(bf, eh, edge_index, W1, b1, W2, b2, W3, b3):
    f32 = jnp.float32
    bf = bf.astype(f32)
    eh = eh.astype(f32)

    m_all = pl.pallas_call(
        _edge_mlp_body,
        grid=(N_EDGES // EBLK,),
        in_specs=[
            pl.BlockSpec((F, EBLK), lambda i: (0, i)),
            pl.BlockSpec((F, EBLK), lambda i: (0, i)),
            pl.BlockSpec((F, F), lambda i: (0, 0)),
            pl.BlockSpec((F, 1), lambda i: (0, 0)),
            pl.BlockSpec((F, F), lambda i: (0, 0)),
            pl.BlockSpec((F, 1), lambda i: (0, 0)),
        ],
        out_specs=pl.BlockSpec((2, EBLK // 4, 128), lambda i: (0, i, 0)),
        out_shape=jax.ShapeDtypeStruct((2, E_PAD // 4, 128), f32),
    )(bf.T, eh.T, W1.astype(f32).T, b1.astype(f32).reshape(F, 1),
      W2.astype(f32).T, b2.astype(f32).reshape(F, 1))
    m_all = m_all.reshape(2, E_PAD, HALF)

    dst = edge_index[1].astype(jnp.int32)
    n_extra = E_PAD - N_EDGES
    pad_dst = N_NODES + (jnp.arange(n_extra, dtype=jnp.int32)
                         % (NPAD - N_NODES))
    dst_perm = (jnp.concatenate([dst, pad_dst])
                .reshape(E_PAD // EBLK, 4, EBLK // 4)
                .transpose(0, 2, 1).reshape(-1))
    dst2 = dst_perm.reshape(E_PAD // 128, 128)

    mesh = plsc.VectorSubcoreMesh(core_axis_name="c", subcore_axis_name="s")
    sums, cnts = pl.kernel(
        _sc_scatter_body,
        out_type=[jax.ShapeDtypeStruct((2, NPAD, HALF), f32),
                  jax.ShapeDtypeStruct((2, NPAD), f32)],
        mesh=mesh,
        scratch_types=[
            pltpu.VMEM((E_CHUNK // 128, 128), jnp.int32),
            pltpu.VMEM((2, QROWS, HALF), f32),
            pltpu.VMEM((128,), f32),
            pltpu.VMEM((NODE_HCHUNK,), f32),
            pltpu.VMEM_SHARED((NPAD, HALF), f32),
            pltpu.VMEM_SHARED((NPAD,), f32),
            pltpu.SemaphoreType.DMA,
            pltpu.SemaphoreType.DMA,
        ],
        compiler_params=pltpu.CompilerParams(use_tc_tiling_on_sc=False),
    )(dst2, m_all)

    out = pl.pallas_call(
        _node_mlp_body,
        grid=(N_NODES // NBLK,),
        in_specs=[
            pl.BlockSpec((2, NBLK, HALF), lambda i: (0, i, 0)),
            pl.BlockSpec((2, NBLK, 1), lambda i: (0, i, 0)),
            pl.BlockSpec((F, F), lambda i: (0, 0)),
            pl.BlockSpec((1, F), lambda i: (0, 0)),
        ],
        out_specs=pl.BlockSpec((NBLK, F), lambda i: (i, 0)),
        out_shape=jax.ShapeDtypeStruct((N_NODES, F), f32),
    )(sums, cnts.reshape(2, NPAD, 1), W3.astype(f32),
      b3.astype(f32).reshape(1, F))
    return out
